# baseline (speedup 1.0000x reference)
import functools

import jax
import jax.numpy as jnp
from jax import lax
from jax.experimental import pallas as pl
from jax.experimental.pallas import tpu as pltpu
from jax.experimental.pallas import tpu_sc as plsc

NC = 2
NS = 16
NW = NC * NS

CHUNK = 40
SETS = 2
HALVES = 1


def _pack_halves(v):
    d = v.shape[-1]
    h = d // 2
    bits = lax.bitcast_convert_type(v.astype(jnp.bfloat16), jnp.uint16)
    lo = bits[:, :h].astype(jnp.uint32)
    hi = bits[:, h:].astype(jnp.uint32)
    return lax.bitcast_convert_type(lo | (hi << 16), jnp.int32)


def _tables_body(x_ref, w1_ref, w2_ref, wrbf_ref, w3_ref, y_ref, z_ref, w3c_ref):
    x = x_ref[...]
    y_ref[...] = _pack_halves(
        jnp.dot(x, w1_ref[...], preferred_element_type=jnp.float32))
    z_ref[...] = _pack_halves(
        jnp.dot(x, w2_ref[...], preferred_element_type=jnp.float32))
    w3c_ref[...] = jnp.dot(wrbf_ref[...], w3_ref[...],
                           preferred_element_type=jnp.float32)


def _make_tables(x, w1, w2, wrbf, w3):
    n, d = x.shape
    return pl.pallas_call(
        _tables_body,
        out_shape=(
            jax.ShapeDtypeStruct((n, d // 2), jnp.int32),
            jax.ShapeDtypeStruct((n, d // 2), jnp.int32),
            jax.ShapeDtypeStruct((wrbf.shape[0], d), jnp.float32),
        ),
    )(x, w1, w2, wrbf, w3)


def _sc_gather_body(epw, d2, y_hbm, z_hbm, idxj_hbm, idxi_hbm, g_hbm,
                    idxja, idxia, y_sp, z_sp, *bufs):
    d = 2 * d2
    sid = lax.axis_index("s")
    wid = sid * NC + lax.axis_index("c")
    base0 = wid * epw
    nchunks = epw // CHUNK
    yj = bufs[0:SETS]
    zi = bufs[SETS:2 * SETS]
    sm = bufs[2 * SETS:3 * SETS]
    gsem = bufs[3 * SETS:4 * SETS]
    wsem = bufs[4 * SETS:5 * SETS]

    @pl.when(sid == 0)
    def _stage():
        pltpu.sync_copy(y_hbm, y_sp)
        pltpu.sync_copy(z_hbm, z_sp)

    plsc.subcore_barrier()

    pltpu.sync_copy(idxj_hbm.at[pl.ds(base0, epw)], idxja)
    pltpu.sync_copy(idxi_hbm.at[pl.ds(base0, epw)], idxia)

    def start_chunk(chunk, b):
        off = chunk * CHUNK
        pltpu.async_copy(y_sp.at[idxja.at[pl.ds(off, CHUNK)]], yj[b], gsem[b])
        pltpu.async_copy(z_sp.at[idxia.at[pl.ds(off, CHUNK)]], zi[b], gsem[b])

    for b in range(SETS):
        start_chunk(b, b)

    hi_mask = jnp.uint32(0xFFFF0000)
    half = jnp.uint32(0x8000)

    @pl.loop(0, nchunks, step=SETS)
    def _chunks(c):
        for b in range(SETS):
            chunk = c + b
            base = base0 + chunk * CHUNK
            pltpu.make_async_copy(y_sp.at[idxja.at[pl.ds(0, CHUNK)]],
                                  yj[b], gsem[b]).wait()
            pltpu.make_async_copy(z_sp.at[idxia.at[pl.ds(0, CHUNK)]],
                                  zi[b], gsem[b]).wait()

            @pl.when(chunk >= SETS)
            def _drain():
                pltpu.make_async_copy(
                    sm[b], g_hbm.at[pl.ds(0, CHUNK * d)], wsem[b]).wait()

            yjb, zib, smb = yj[b], zi[b], sm[b]

            @plsc.parallel_loop(0, CHUNK, unroll=2)
            def _rows(i):
                for k in range(d2 // 16):
                    sl = pl.ds(k * 16, 16)
                    wa = plsc.bitcast(yjb[i, sl], jnp.uint32)
                    wc = plsc.bitcast(zib[i, sl], jnp.uint32)
                    slo = (plsc.bitcast(wa << 16, jnp.float32)
                           + plsc.bitcast(wc << 16, jnp.float32))
                    shi = (plsc.bitcast(wa & hi_mask, jnp.float32)
                           + plsc.bitcast(wc & hi_mask, jnp.float32))
                    ulo = plsc.bitcast(slo, jnp.uint32) + half
                    uhi = plsc.bitcast(shi, jnp.uint32) + half
                    packed = (ulo >> 16) | (uhi & hi_mask)
                    smb[pl.ds(i * d + k * 32, 32)] = plsc.bitcast(
                        packed, jnp.bfloat16)

            pltpu.async_copy(smb, g_hbm.at[pl.ds(base * d, CHUNK * d)],
                             wsem[b])

            @pl.when(chunk + SETS < nchunks)
            def _prefetch():
                start_chunk(chunk + SETS, b)

    for b in range(SETS):
        pltpu.make_async_copy(sm[b], g_hbm.at[pl.ds(0, CHUNK * d)],
                              wsem[b]).wait()


def _sc_gather(y32, z32, idx_j, idx_i):
    e = idx_j.shape[0]
    n_nodes = y32.shape[0]
    d2 = y32.shape[1]
    d = 2 * d2
    assert e % (NW * CHUNK) == 0 and (e // (NW * CHUNK)) % SETS == 0
    epw = e // NW
    mesh = plsc.VectorSubcoreMesh(core_axis_name="c", subcore_axis_name="s",
                                  num_cores=NC, num_subcores=NS)
    kern = pl.kernel(
        functools.partial(_sc_gather_body, epw, d2),
        out_type=jax.ShapeDtypeStruct((e * d,), jnp.bfloat16),
        mesh=mesh,
        compiler_params=pltpu.CompilerParams(needs_layout_passes=False,
                                             use_tc_tiling_on_sc=False),
        scratch_types=(
            [pltpu.VMEM((epw,), jnp.int32)] * 2
            + [pltpu.VMEM_SHARED((n_nodes, d2), jnp.int32)] * 2
            + [pltpu.VMEM((CHUNK, d2), jnp.int32)] * (2 * SETS)
            + [pltpu.VMEM((CHUNK * d,), jnp.bfloat16)] * SETS
            + [pltpu.SemaphoreType.DMA] * (2 * SETS)
        ),
    )
    return kern(y32, z32, idx_j, idx_i)


def _combine_body(block, d, g_ref, rbf_ref, w3c_ref, b_ref, out_ref):
    g = g_ref[...].reshape(block, d).astype(jnp.float32)
    s = (g
         + jnp.dot(rbf_ref[...], w3c_ref[...],
                   preferred_element_type=jnp.float32)
         + b_ref[...])
    out_ref[...] = s * jax.nn.sigmoid(s)


def _combine_body_alias(block, d, g_ref, rbf_ref, w3c_ref, b_ref, prev_ref,
                        out_ref):
    del prev_ref
    _combine_body(block, d, g_ref, rbf_ref, w3c_ref, b_ref, out_ref)


def _combine_half(g1d_half, rbf, w3c, b2d, block, half, prev_out):
    e, nrad = rbf.shape
    d = g1d_half.shape[0] * HALVES // e
    eh = e // HALVES
    assert eh % block == 0
    base = half * (eh // block)
    grid = (eh // block,)
    in_specs = [
        pl.BlockSpec((block * d,), lambda i: (i,)),
        pl.BlockSpec((block, nrad), lambda i: (i + base, 0)),
        pl.BlockSpec((nrad, d), lambda i: (0, 0)),
        pl.BlockSpec((1, d), lambda i: (0, 0)),
    ]
    args = [g1d_half, rbf, w3c, b2d]
    kwargs = {}
    if prev_out is not None:
        in_specs.append(pl.BlockSpec(memory_space=pl.ANY))
        args.append(prev_out)
        kwargs = dict(input_output_aliases={4: 0})
    body = functools.partial(_combine_body, block, d)
    if prev_out is not None:
        body = functools.partial(_combine_body_alias, block, d)
    return pl.pallas_call(
        body,
        grid=grid,
        in_specs=in_specs,
        out_specs=pl.BlockSpec((block, d), lambda i: (i + base, 0)),
        out_shape=jax.ShapeDtypeStruct((e, d), jnp.float32),
        **kwargs,
    )(*args)


def kernel(x, rbf, idx_i, idx_j, W_rbf, W_edge, b_edge):
    d = x.shape[1]
    w1 = W_edge[:d]
    w2 = W_edge[d:2 * d]
    w3 = W_edge[2 * d:]
    idx_i = idx_i.astype(jnp.int32)
    idx_j = idx_j.astype(jnp.int32)

    perm = jnp.concatenate([jnp.arange(0, d, 2), jnp.arange(1, d, 2)])
    y32, z32, w3c = _make_tables(x, w1[:, perm], w2[:, perm], W_rbf, w3)
    e = idx_j.shape[0]
    eh = e // HALVES
    b2d = b_edge.reshape(1, d)
    out = None
    for h in range(HALVES):
        sl = slice(h * eh, (h + 1) * eh)
        g1d = _sc_gather(y32, z32, idx_j[sl], idx_i[sl])
        out = _combine_half(g1d, rbf, w3c, b2d, 4000, h, out)
    return out


# speedup vs baseline: 2.0945x; 2.0945x over previous
import functools

import jax
import jax.numpy as jnp
from jax import lax
from jax.experimental import pallas as pl
from jax.experimental.pallas import tpu as pltpu
from jax.experimental.pallas import tpu_sc as plsc

NC = 2
NS = 16
NW = NC * NS

CHUNK = 40
SETS = 2
HALVES = 1


def _pack_halves(v):
    d = v.shape[-1]
    h = d // 2
    bits = lax.bitcast_convert_type(v.astype(jnp.bfloat16), jnp.uint16)
    lo = bits[:, :h].astype(jnp.uint32)
    hi = bits[:, h:].astype(jnp.uint32)
    return lax.bitcast_convert_type(lo | (hi << 16), jnp.int32)


def _tables_body(x_ref, w1_ref, w2_ref, wrbf_ref, w3_ref, y_ref, z_ref, w3c_ref):
    x = x_ref[...]
    y_ref[...] = _pack_halves(
        jnp.dot(x, w1_ref[...], preferred_element_type=jnp.float32))
    z_ref[...] = _pack_halves(
        jnp.dot(x, w2_ref[...], preferred_element_type=jnp.float32))
    w3c_ref[...] = jnp.dot(wrbf_ref[...], w3_ref[...],
                           preferred_element_type=jnp.float32)


def _make_tables(x, w1, w2, wrbf, w3):
    n, d = x.shape
    return pl.pallas_call(
        _tables_body,
        out_shape=(
            jax.ShapeDtypeStruct((n, d // 2), jnp.int32),
            jax.ShapeDtypeStruct((n, d // 2), jnp.int32),
            jax.ShapeDtypeStruct((wrbf.shape[0], d), jnp.float32),
        ),
    )(x, w1, w2, wrbf, w3)


def _sc_gather_body(epw, d2, y_hbm, z_hbm, idxj_hbm, idxi_hbm, g_hbm,
                    idxja, idxia, y_sp, z_sp, *bufs):
    d = 2 * d2
    sid = lax.axis_index("s")
    wid = sid * NC + lax.axis_index("c")
    base0 = wid * epw
    nchunks = epw // CHUNK
    yj = bufs[0:SETS]
    zi = bufs[SETS:2 * SETS]
    sm = bufs[2 * SETS:3 * SETS]
    gsem = bufs[3 * SETS:4 * SETS]
    wsem = bufs[4 * SETS:5 * SETS]

    @pl.when(sid == 0)
    def _stage():
        pltpu.sync_copy(y_hbm, y_sp)
        pltpu.sync_copy(z_hbm, z_sp)

    plsc.subcore_barrier()

    pltpu.sync_copy(idxj_hbm.at[pl.ds(base0, epw)], idxja)
    pltpu.sync_copy(idxi_hbm.at[pl.ds(base0, epw)], idxia)

    def start_chunk(chunk, b):
        off = chunk * CHUNK
        pltpu.async_copy(y_sp.at[idxja.at[pl.ds(off, CHUNK)]], yj[b], gsem[b])
        pltpu.async_copy(z_sp.at[idxia.at[pl.ds(off, CHUNK)]], zi[b], gsem[b])

    for b in range(SETS):
        start_chunk(b, b)

    hi_mask = jnp.uint32(0xFFFF0000)
    half = jnp.uint32(0x8000)

    @pl.loop(0, nchunks, step=SETS)
    def _chunks(c):
        for b in range(SETS):
            chunk = c + b
            base = base0 + chunk * CHUNK
            pltpu.make_async_copy(y_sp.at[idxja.at[pl.ds(0, CHUNK)]],
                                  yj[b], gsem[b]).wait()
            pltpu.make_async_copy(z_sp.at[idxia.at[pl.ds(0, CHUNK)]],
                                  zi[b], gsem[b]).wait()

            @pl.when(chunk >= SETS)
            def _drain():
                pltpu.make_async_copy(
                    sm[b], g_hbm.at[pl.ds(0, CHUNK * d)], wsem[b]).wait()

            yjb, zib, smb = yj[b], zi[b], sm[b]

            @plsc.parallel_loop(0, CHUNK, unroll=2)
            def _rows(i):
                for k in range(d2 // 16):
                    sl = pl.ds(k * 16, 16)
                    wa = plsc.bitcast(yjb[i, sl], jnp.uint32)
                    wc = plsc.bitcast(zib[i, sl], jnp.uint32)
                    slo = (plsc.bitcast(wa << 16, jnp.float32)
                           + plsc.bitcast(wc << 16, jnp.float32))
                    shi = (plsc.bitcast(wa & hi_mask, jnp.float32)
                           + plsc.bitcast(wc & hi_mask, jnp.float32))
                    smb[pl.ds(i * d + k * 16, 16)] = slo
                    smb[pl.ds(i * d + d2 + k * 16, 16)] = shi

            pltpu.async_copy(smb, g_hbm.at[pl.ds(base * d, CHUNK * d)],
                             wsem[b])

            @pl.when(chunk + SETS < nchunks)
            def _prefetch():
                start_chunk(chunk + SETS, b)

    for b in range(SETS):
        pltpu.make_async_copy(sm[b], g_hbm.at[pl.ds(0, CHUNK * d)],
                              wsem[b]).wait()


def _sc_gather(y32, z32, idx_j, idx_i):
    e = idx_j.shape[0]
    n_nodes = y32.shape[0]
    d2 = y32.shape[1]
    d = 2 * d2
    assert e % (NW * CHUNK) == 0 and (e // (NW * CHUNK)) % SETS == 0
    epw = e // NW
    mesh = plsc.VectorSubcoreMesh(core_axis_name="c", subcore_axis_name="s",
                                  num_cores=NC, num_subcores=NS)
    kern = pl.kernel(
        functools.partial(_sc_gather_body, epw, d2),
        out_type=jax.ShapeDtypeStruct((e * d,), jnp.float32),
        mesh=mesh,
        compiler_params=pltpu.CompilerParams(needs_layout_passes=False,
                                             use_tc_tiling_on_sc=False),
        scratch_types=(
            [pltpu.VMEM((epw,), jnp.int32)] * 2
            + [pltpu.VMEM_SHARED((n_nodes, d2), jnp.int32)] * 2
            + [pltpu.VMEM((CHUNK, d2), jnp.int32)] * (2 * SETS)
            + [pltpu.VMEM((CHUNK * d,), jnp.float32)] * SETS
            + [pltpu.SemaphoreType.DMA] * (2 * SETS)
        ),
    )
    return kern(y32, z32, idx_j, idx_i)


def _combine_body(block, d, g_ref, rbf_ref, w3c_ref, b_ref, out_ref):
    g = g_ref[...].reshape(block, d)
    s = (g
         + jnp.dot(rbf_ref[...], w3c_ref[...],
                   preferred_element_type=jnp.float32)
         + b_ref[...])
    out_ref[...] = s * jax.nn.sigmoid(s)


def _combine_body_alias(block, d, g_ref, rbf_ref, w3c_ref, b_ref, prev_ref,
                        out_ref):
    del prev_ref
    _combine_body(block, d, g_ref, rbf_ref, w3c_ref, b_ref, out_ref)


def _combine_half(g1d_half, rbf, w3c, b2d, block, half, prev_out):
    e, nrad = rbf.shape
    d = g1d_half.shape[0] * HALVES // e
    eh = e // HALVES
    assert eh % block == 0
    base = half * (eh // block)
    grid = (eh // block,)
    in_specs = [
        pl.BlockSpec((block * d,), lambda i: (i,)),
        pl.BlockSpec((block, nrad), lambda i: (i + base, 0)),
        pl.BlockSpec((nrad, d), lambda i: (0, 0)),
        pl.BlockSpec((1, d), lambda i: (0, 0)),
    ]
    args = [g1d_half, rbf, w3c, b2d]
    kwargs = {}
    if prev_out is not None:
        in_specs.append(pl.BlockSpec(memory_space=pl.ANY))
        args.append(prev_out)
        kwargs = dict(input_output_aliases={4: 0})
    body = functools.partial(_combine_body, block, d)
    if prev_out is not None:
        body = functools.partial(_combine_body_alias, block, d)
    return pl.pallas_call(
        body,
        grid=grid,
        in_specs=in_specs,
        out_specs=pl.BlockSpec((block, d), lambda i: (i + base, 0)),
        out_shape=jax.ShapeDtypeStruct((e, d), jnp.float32),
        **kwargs,
    )(*args)


def kernel(x, rbf, idx_i, idx_j, W_rbf, W_edge, b_edge):
    d = x.shape[1]
    w1 = W_edge[:d]
    w2 = W_edge[d:2 * d]
    w3 = W_edge[2 * d:]
    idx_i = idx_i.astype(jnp.int32)
    idx_j = idx_j.astype(jnp.int32)

    y32, z32, w3c = _make_tables(x, w1, w2, W_rbf, w3)
    e = idx_j.shape[0]
    eh = e // HALVES
    b2d = b_edge.reshape(1, d)
    out = None
    for h in range(HALVES):
        sl = slice(h * eh, (h + 1) * eh)
        g1d = _sc_gather(y32, z32, idx_j[sl], idx_i[sl])
        out = _combine_half(g1d, rbf, w3c, b2d, 8000, h, out)
    return out
